# baseline (speedup 1.0000x reference)
import functools

import jax
import jax.numpy as jnp
from jax import lax
from jax.experimental import pallas as pl
from jax.experimental.pallas import tpu as pltpu
from jax.experimental.pallas import tpu_sc as plsc

B = 16384
EMB = 128
EPS = 1e-5

NUM_WORKERS = 32
ROWS_PER_W = B // NUM_WORKERS
CHUNK = 128
NCHUNK = ROWS_PER_W // CHUNK


HALF = ROWS_PER_W // 2


def _sc_gather_body(uidx_hbm, iidx_hbm, utab_hbm, itab_hbm,
                    x_out, uidx_v, iidx_v, rows_a, rows_b, gsem, wsem):
    wid = lax.axis_index("s") * 2 + lax.axis_index("c")
    base = wid * ROWS_PER_W
    pltpu.sync_copy(uidx_hbm.at[pl.ds(base, ROWS_PER_W)], uidx_v)
    pltpu.sync_copy(iidx_hbm.at[pl.ds(base, ROWS_PER_W)], iidx_v)
    gathers = [pltpu.async_copy(
        utab_hbm.at[uidx_v.at[pl.ds(j * CHUNK, CHUNK)]],
        rows_a.at[pl.ds(j * CHUNK, CHUNK)], gsem) for j in range(NCHUNK)]
    for g in gathers:
        g.wait()
    w_a = pltpu.async_copy(
        rows_a, x_out.at[pl.ds(base, ROWS_PER_W), pl.ds(0, EMB)], wsem)
    gathers = [pltpu.async_copy(
        itab_hbm.at[iidx_v.at[pl.ds(j * CHUNK, CHUNK)]],
        rows_b.at[pl.ds(j * CHUNK, CHUNK)], gsem) for j in range(2)]
    for g in gathers:
        g.wait()
    w_a.wait()
    w_b = pltpu.async_copy(
        rows_b, x_out.at[pl.ds(base, HALF), pl.ds(EMB, EMB)], wsem)
    gathers = [pltpu.async_copy(
        itab_hbm.at[iidx_v.at[pl.ds((2 + j) * CHUNK, CHUNK)]],
        rows_a.at[pl.ds(j * CHUNK, CHUNK)], gsem) for j in range(2)]
    for g in gathers:
        g.wait()
    w_b.wait()
    pltpu.sync_copy(rows_a.at[pl.ds(0, HALF)],
                    x_out.at[pl.ds(base + HALF, HALF), pl.ds(EMB, EMB)])


@functools.cache
def _make_gather():
    return pl.kernel(
        _sc_gather_body,
        mesh=plsc.VectorSubcoreMesh(core_axis_name="c", subcore_axis_name="s"),
        out_type=jax.ShapeDtypeStruct((B, 2 * EMB), jnp.float32),
        scratch_types=[pltpu.VMEM((ROWS_PER_W,), jnp.int32),
                       pltpu.VMEM((ROWS_PER_W,), jnp.int32),
                       pltpu.VMEM((ROWS_PER_W, EMB), jnp.float32),
                       pltpu.VMEM((HALF, EMB), jnp.float32),
                       pltpu.SemaphoreType.DMA,
                       pltpu.SemaphoreType.DMA],
    )


NB = 8
BB = B // NB


def _mlp_body(x2, W1, b1, g1, beta1, W23, b23, g3, beta3, Wo, bo, out,
              x_all, h_all, s1, ss1, a1c1, s3, ss3, a3c3):
    i = pl.program_id(0)

    @pl.when(i < NB)
    def _phase_a():
        xb = jnp.dot(x2[...], W1[...]) + b1[...]
        x_all[pl.ds(i * BB, BB), :] = xb
        ps = jnp.sum(xb, axis=0, keepdims=True)
        pss = jnp.sum(xb * xb, axis=0, keepdims=True)

        @pl.when(i == 0)
        def _():
            s1[...] = ps
            ss1[...] = pss

        @pl.when(i > 0)
        def _():
            s1[...] += ps
            ss1[...] += pss

    @pl.when((i >= NB) & (i < 2 * NB))
    def _phase_b():
        j = i - NB

        @pl.when(i == NB)
        def _():
            mu = s1[...] / B
            var = ss1[...] / B - mu * mu
            a = g1[...] * lax.rsqrt(var + EPS)
            a1c1[0:1, :] = a
            a1c1[1:2, :] = beta1[...] - a * mu

        xb = x_all[pl.ds(j * BB, BB), :]
        y = jnp.maximum(a1c1[0:1, :] * xb + a1c1[1:2, :], 0.0)
        hb = jnp.dot(y, W23[...]) + b23[...]
        h_all[pl.ds(j * BB, BB), :] = hb
        ps = jnp.sum(hb, axis=0, keepdims=True)
        pss = jnp.sum(hb * hb, axis=0, keepdims=True)

        @pl.when(i == NB)
        def _():
            s3[...] = ps
            ss3[...] = pss

        @pl.when(i > NB)
        def _():
            s3[...] += ps
            ss3[...] += pss

    @pl.when(i >= 2 * NB)
    def _phase_c():
        j = i - 2 * NB

        @pl.when(i == 2 * NB)
        def _():
            mu = s3[...] / B
            var = ss3[...] / B - mu * mu
            a = g3[...] * lax.rsqrt(var + EPS)
            a3c3[0:1, :] = a
            a3c3[1:2, :] = beta3[...] - a * mu

        hb = h_all[pl.ds(j * BB, BB), :]
        hh = jnp.maximum(a3c3[0:1, :] * hb + a3c3[1:2, :], 0.0)
        out[...] = (jnp.dot(hh, Wo[...]) + bo[...]).reshape(BB)


_mlp = pl.pallas_call(
    _mlp_body,
    grid=(3 * NB,),
    in_specs=[
        pl.BlockSpec((BB, 2 * EMB), lambda i: (jnp.minimum(i, NB - 1), 0)),
        pl.BlockSpec((2 * EMB, EMB), lambda i: (0, 0)),
        pl.BlockSpec((1, EMB), lambda i: (0, 0)),
        pl.BlockSpec((1, EMB), lambda i: (0, 0)),
        pl.BlockSpec((1, EMB), lambda i: (0, 0)),
        pl.BlockSpec((EMB, 32), lambda i: (0, 0)),
        pl.BlockSpec((1, 32), lambda i: (0, 0)),
        pl.BlockSpec((1, 32), lambda i: (0, 0)),
        pl.BlockSpec((1, 32), lambda i: (0, 0)),
        pl.BlockSpec((32, 1), lambda i: (0, 0)),
        pl.BlockSpec((1, 1), lambda i: (0, 0)),
    ],
    out_specs=pl.BlockSpec((BB,), lambda i: (jnp.maximum(i - 2 * NB, 0),)),
    out_shape=jax.ShapeDtypeStruct((B,), jnp.float32),
    scratch_shapes=[
        pltpu.VMEM((B, EMB), jnp.float32),
        pltpu.VMEM((B, 32), jnp.float32),
        pltpu.VMEM((1, EMB), jnp.float32),
        pltpu.VMEM((1, EMB), jnp.float32),
        pltpu.VMEM((2, EMB), jnp.float32),
        pltpu.VMEM((1, 32), jnp.float32),
        pltpu.VMEM((1, 32), jnp.float32),
        pltpu.VMEM((2, 32), jnp.float32),
    ],
)


def _mlp_out_2d(*args):
    return _mlp(*args).reshape(B, 1)


def kernel(user_input, item_input, user_table, item_table,
           uW1, ub1, ug1, ubeta1, uW2, ub2,
           iW1, ib1, ig1, ibeta1, iW2, ib2,
           W3, b3, g3, beta3, Wo, bo):
    uidx = user_input.astype(jnp.int32)
    iidx = item_input.astype(jnp.int32)
    x2 = _make_gather()(uidx, iidx, user_table, item_table)

    W1 = jnp.concatenate(
        [jnp.concatenate([uW1, jnp.zeros_like(uW1)], axis=1),
         jnp.concatenate([jnp.zeros_like(iW1), iW1], axis=1)], axis=0)
    W2 = jnp.concatenate(
        [jnp.concatenate([uW2, jnp.zeros_like(uW2)], axis=1),
         jnp.concatenate([jnp.zeros_like(iW2), iW2], axis=1)], axis=0)
    W23 = W2 @ W3
    b23 = jnp.concatenate([ub2, ib2]) @ W3 + b3
    r = lambda v: v.reshape(1, -1)
    return _mlp_out_2d(x2, W1, r(jnp.concatenate([ub1, ib1])),
                r(jnp.concatenate([ug1, ig1])),
                r(jnp.concatenate([ubeta1, ibeta1])),
                W23, r(b23), r(g3), r(beta3), Wo, r(bo))


# speedup vs baseline: 1.0247x; 1.0247x over previous
import functools

import jax
import jax.numpy as jnp
from jax import lax
from jax.experimental import pallas as pl
from jax.experimental.pallas import tpu as pltpu
from jax.experimental.pallas import tpu_sc as plsc

B = 16384
EMB = 128
EPS = 1e-5

NUM_WORKERS = 32
ROWS_PER_W = B // NUM_WORKERS
CHUNK = 128
NCHUNK = ROWS_PER_W // CHUNK


HALF = ROWS_PER_W // 2


def _sc_gather_body(uidx_hbm, iidx_hbm, utab_hbm, itab_hbm,
                    x_out, uidx_v, iidx_v, rows_a, rows_b, gsem, wsem):
    wid = lax.axis_index("s") * 2 + lax.axis_index("c")
    base = wid * ROWS_PER_W
    pltpu.sync_copy(uidx_hbm.at[pl.ds(base, ROWS_PER_W)], uidx_v)
    pltpu.sync_copy(iidx_hbm.at[pl.ds(base, ROWS_PER_W)], iidx_v)
    gathers = [pltpu.async_copy(
        utab_hbm.at[uidx_v.at[pl.ds(j * CHUNK, CHUNK)]],
        rows_a.at[pl.ds(j * CHUNK, CHUNK)], gsem) for j in range(NCHUNK)]
    for g in gathers:
        g.wait()
    w_a = pltpu.async_copy(
        rows_a, x_out.at[pl.ds(base, ROWS_PER_W), pl.ds(0, EMB)], wsem)
    gathers = [pltpu.async_copy(
        itab_hbm.at[iidx_v.at[pl.ds(j * CHUNK, CHUNK)]],
        rows_b.at[pl.ds(j * CHUNK, CHUNK)], gsem) for j in range(2)]
    for g in gathers:
        g.wait()
    w_a.wait()
    w_b = pltpu.async_copy(
        rows_b, x_out.at[pl.ds(base, HALF), pl.ds(EMB, EMB)], wsem)
    gathers = [pltpu.async_copy(
        itab_hbm.at[iidx_v.at[pl.ds((2 + j) * CHUNK, CHUNK)]],
        rows_a.at[pl.ds(j * CHUNK, CHUNK)], gsem) for j in range(2)]
    for g in gathers:
        g.wait()
    w_b.wait()
    pltpu.sync_copy(rows_a.at[pl.ds(0, HALF)],
                    x_out.at[pl.ds(base + HALF, HALF), pl.ds(EMB, EMB)])


@functools.cache
def _make_gather():
    return pl.kernel(
        _sc_gather_body,
        mesh=plsc.VectorSubcoreMesh(core_axis_name="c", subcore_axis_name="s"),
        out_type=jax.ShapeDtypeStruct((B, 2 * EMB), jnp.float32),
        scratch_types=[pltpu.VMEM((ROWS_PER_W,), jnp.int32),
                       pltpu.VMEM((ROWS_PER_W,), jnp.int32),
                       pltpu.VMEM((ROWS_PER_W, EMB), jnp.float32),
                       pltpu.VMEM((HALF, EMB), jnp.float32),
                       pltpu.SemaphoreType.DMA,
                       pltpu.SemaphoreType.DMA],
    )


NB = 8
BB = B // NB


def _mlp_body(x2_hbm, W1, b1, g1, beta1, W23, b23, g3, beta3, Wo, bo, out,
              x2_buf, x_all, sems):
    def start(k):
        pltpu.make_async_copy(
            x2_hbm.at[pl.ds(k * BB, BB), :], x2_buf.at[k % 2],
            sems.at[k % 2]).start()

    start(0)
    s1 = jnp.zeros((1, EMB), jnp.float32)
    ss1 = jnp.zeros((1, EMB), jnp.float32)
    for k in range(NB):
        if k + 1 < NB:
            start(k + 1)
        pltpu.make_async_copy(
            x2_hbm.at[pl.ds(k * BB, BB), :], x2_buf.at[k % 2],
            sems.at[k % 2]).wait()
        xb = jnp.dot(x2_buf[k % 2], W1[...]) + b1[...]
        x_all[pl.ds(k * BB, BB), :] = xb
        s1 = s1 + jnp.sum(xb, axis=0, keepdims=True)
        ss1 = ss1 + jnp.sum(xb * xb, axis=0, keepdims=True)

    mu = s1 / B
    var = ss1 / B - mu * mu
    a = g1[...] * lax.rsqrt(var + EPS)
    c = beta1[...] - a * mu
    y = jnp.maximum(a * x_all[...] + c, 0.0)
    h = jnp.dot(y, W23[...]) + b23[...]
    mu3 = jnp.mean(h, axis=0, keepdims=True)
    var3 = jnp.mean(h * h, axis=0, keepdims=True) - mu3 * mu3
    a3 = g3[...] * lax.rsqrt(var3 + EPS)
    c3 = beta3[...] - a3 * mu3
    hh = jnp.maximum(a3 * h + c3, 0.0)
    out[...] = (jnp.dot(hh, Wo[...]) + bo[...]).reshape(B)


_mlp = pl.pallas_call(
    _mlp_body,
    in_specs=[pl.BlockSpec(memory_space=pltpu.MemorySpace.HBM)]
             + [pl.BlockSpec()] * 10,
    out_shape=jax.ShapeDtypeStruct((B,), jnp.float32),
    scratch_shapes=[
        pltpu.VMEM((2, BB, 2 * EMB), jnp.float32),
        pltpu.VMEM((B, EMB), jnp.float32),
        pltpu.SemaphoreType.DMA((2,)),
    ],
)


def _mlp_out_2d(*args):
    return _mlp(*args).reshape(B, 1)


def kernel(user_input, item_input, user_table, item_table,
           uW1, ub1, ug1, ubeta1, uW2, ub2,
           iW1, ib1, ig1, ibeta1, iW2, ib2,
           W3, b3, g3, beta3, Wo, bo):
    uidx = user_input.astype(jnp.int32)
    iidx = item_input.astype(jnp.int32)
    x2 = _make_gather()(uidx, iidx, user_table, item_table)

    W1 = jnp.concatenate(
        [jnp.concatenate([uW1, jnp.zeros_like(uW1)], axis=1),
         jnp.concatenate([jnp.zeros_like(iW1), iW1], axis=1)], axis=0)
    W2 = jnp.concatenate(
        [jnp.concatenate([uW2, jnp.zeros_like(uW2)], axis=1),
         jnp.concatenate([jnp.zeros_like(iW2), iW2], axis=1)], axis=0)
    W23 = W2 @ W3
    b23 = jnp.concatenate([ub2, ib2]) @ W3 + b3
    r = lambda v: v.reshape(1, -1)
    return _mlp_out_2d(x2, W1, r(jnp.concatenate([ub1, ib1])),
                r(jnp.concatenate([ug1, ig1])),
                r(jnp.concatenate([ubeta1, ibeta1])),
                W23, r(b23), r(g3), r(beta3), Wo, r(bo))


# speedup vs baseline: 1.0322x; 1.0073x over previous
import functools

import jax
import jax.numpy as jnp
from jax import lax
from jax.experimental import pallas as pl
from jax.experimental.pallas import tpu as pltpu
from jax.experimental.pallas import tpu_sc as plsc

B = 16384
EMB = 128
EPS = 1e-5

NUM_WORKERS = 32
ROWS_PER_W = B // NUM_WORKERS
CHUNK = 128
NCHUNK = ROWS_PER_W // CHUNK


HALF = ROWS_PER_W // 2


def _sc_gather_body(uidx_hbm, iidx_hbm, utab_hbm, itab_hbm,
                    x_out, uidx_v, iidx_v, rows_a, rows_b, gsem, wsem):
    wid = lax.axis_index("s") * 2 + lax.axis_index("c")
    base = wid * ROWS_PER_W
    pltpu.sync_copy(uidx_hbm.at[pl.ds(base, ROWS_PER_W)], uidx_v)
    pltpu.sync_copy(iidx_hbm.at[pl.ds(base, ROWS_PER_W)], iidx_v)
    gathers = [pltpu.async_copy(
        utab_hbm.at[uidx_v.at[pl.ds(j * CHUNK, CHUNK)]],
        rows_a.at[pl.ds(j * CHUNK, CHUNK)], gsem) for j in range(NCHUNK)]
    for g in gathers:
        g.wait()
    w_a = pltpu.async_copy(
        rows_a, x_out.at[pl.ds(base, ROWS_PER_W), pl.ds(0, EMB)], wsem)
    gathers = [pltpu.async_copy(
        itab_hbm.at[iidx_v.at[pl.ds(j * CHUNK, CHUNK)]],
        rows_b.at[pl.ds(j * CHUNK, CHUNK)], gsem) for j in range(2)]
    for g in gathers:
        g.wait()
    w_a.wait()
    w_b = pltpu.async_copy(
        rows_b, x_out.at[pl.ds(base, HALF), pl.ds(EMB, EMB)], wsem)
    gathers = [pltpu.async_copy(
        itab_hbm.at[iidx_v.at[pl.ds((2 + j) * CHUNK, CHUNK)]],
        rows_a.at[pl.ds(j * CHUNK, CHUNK)], gsem) for j in range(2)]
    for g in gathers:
        g.wait()
    w_b.wait()
    pltpu.sync_copy(rows_a.at[pl.ds(0, HALF)],
                    x_out.at[pl.ds(base + HALF, HALF), pl.ds(EMB, EMB)])


@functools.cache
def _make_gather():
    return pl.kernel(
        _sc_gather_body,
        mesh=plsc.VectorSubcoreMesh(core_axis_name="c", subcore_axis_name="s"),
        out_type=jax.ShapeDtypeStruct((B, 2 * EMB), jnp.float32),
        scratch_types=[pltpu.VMEM((ROWS_PER_W,), jnp.int32),
                       pltpu.VMEM((ROWS_PER_W,), jnp.int32),
                       pltpu.VMEM((ROWS_PER_W, EMB), jnp.float32),
                       pltpu.VMEM((HALF, EMB), jnp.float32),
                       pltpu.SemaphoreType.DMA,
                       pltpu.SemaphoreType.DMA],
    )


def _bn_fold(x, g, beta):
    mu = jnp.mean(x, axis=0, keepdims=True)
    var = jnp.mean(x * x, axis=0, keepdims=True) - mu * mu
    a = g * lax.rsqrt(var + EPS)
    c = beta - a * mu
    return jnp.maximum(a * x + c, 0.0)


def _mlp_body(x2, W1, b1, g1, beta1, W23, b23, g3, beta3, Wo, bo, out):
    x = jnp.dot(x2[...], W1[...]) + b1[...]
    y = _bn_fold(x, g1[...], beta1[...])
    h = jnp.dot(y, W23[...]) + b23[...]
    hh = _bn_fold(h, g3[...], beta3[...])
    out[...] = (jnp.dot(hh, Wo[...]) + bo[...]).reshape(B)


_mlp = pl.pallas_call(
    _mlp_body,
    out_shape=jax.ShapeDtypeStruct((B,), jnp.float32),
)


def _mlp_out_2d(*args):
    return _mlp(*args).reshape(B, 1)


def kernel(user_input, item_input, user_table, item_table,
           uW1, ub1, ug1, ubeta1, uW2, ub2,
           iW1, ib1, ig1, ibeta1, iW2, ib2,
           W3, b3, g3, beta3, Wo, bo):
    uidx = user_input.astype(jnp.int32)
    iidx = item_input.astype(jnp.int32)
    x2 = _make_gather()(uidx, iidx, user_table, item_table)

    W1 = jnp.concatenate(
        [jnp.concatenate([uW1, jnp.zeros_like(uW1)], axis=1),
         jnp.concatenate([jnp.zeros_like(iW1), iW1], axis=1)], axis=0)
    W2 = jnp.concatenate(
        [jnp.concatenate([uW2, jnp.zeros_like(uW2)], axis=1),
         jnp.concatenate([jnp.zeros_like(iW2), iW2], axis=1)], axis=0)
    W23 = W2 @ W3
    b23 = jnp.concatenate([ub2, ib2]) @ W3 + b3
    r = lambda v: v.reshape(1, -1)
    return _mlp_out_2d(x2, W1, r(jnp.concatenate([ub1, ib1])),
                r(jnp.concatenate([ug1, ig1])),
                r(jnp.concatenate([ubeta1, ibeta1])),
                W23, r(b23), r(g3), r(beta3), Wo, r(bo))
